# Initial kernel scaffold; baseline (speedup 1.0000x reference)
import jax
import jax.numpy as jnp
from jax.experimental import pallas as pl


def kernel(FM, rois):
    raise NotImplementedError("write your pallas kernel here")


# speedup vs baseline: 4.6686x; 4.6686x over previous
import functools

import jax
import jax.numpy as jnp
from jax import lax
from jax.experimental import pallas as pl
from jax.experimental.pallas import tpu as pltpu
from jax.experimental.pallas import tpu_sc as plsc

_NT = 21
_RHW = 7
_H = 64
_W = 64
_C = _NT * _RHW * _RHW
_NB = _RHW * _RHW
_NR = 128
_LANES = 16
_NCORES = 2
_NSUB = 16
_NWORK = _NCORES * _NSUB
_CPW = -(-_C // _NWORK)


def _ii_body(fm_ref, out_ref):
    cb = fm_ref.shape[0]
    a = fm_ref[...]
    r = lax.broadcasted_iota(jnp.int32, (_W, _W), 0)
    s = lax.broadcasted_iota(jnp.int32, (_W, _W), 1)
    u = (r <= s).astype(jnp.float32)
    b1 = jnp.dot(a.reshape(cb * _H, _W), u,
                 precision=lax.Precision.HIGHEST).reshape(cb, _H, _W)
    b1t = jnp.swapaxes(b1, 1, 2)
    b2 = jnp.dot(b1t.reshape(cb * _W, _H), u,
                 precision=lax.Precision.HIGHEST)
    out_ref[...] = b2.reshape(1, cb * _W, _H)


def _idx_body(roist_ref, idx_ref, wgt_ref):
    x1 = jnp.round(roist_ref[0:1, :])
    y1 = jnp.round(roist_ref[1:2, :])
    x2 = jnp.round(roist_ref[2:3, :])
    y2 = jnp.round(roist_ref[3:4, :])
    bin_w = jnp.maximum(x2 - x1 + 1.0, 1.0) / _RHW
    bin_h = jnp.maximum(y2 - y1 + 1.0, 1.0) / _RHW
    p = lax.broadcasted_iota(jnp.int32, (_RHW, _NR), 0).astype(jnp.float32)
    hs = jnp.clip(jnp.floor(p * bin_h) + y1, 0.0, float(_H))
    he = jnp.clip(jnp.ceil((p + 1.0) * bin_h) + y1, 0.0, float(_H))
    ws = jnp.clip(jnp.floor(p * bin_w) + x1, 0.0, float(_W))
    we = jnp.clip(jnp.ceil((p + 1.0) * bin_w) + x1, 0.0, float(_W))
    invc = 1.0 / jnp.maximum((he - hs)[:, None, :] * (we - ws)[None, :, :], 1.0)
    corners = ((he, we, 1.0), (hs, we, -1.0), (he, ws, -1.0), (hs, ws, 1.0))
    for k, (ya, xb, sign) in enumerate(corners):
        yi = ya.astype(jnp.int32)[:, None, :]
        xi = xb.astype(jnp.int32)[None, :, :]
        valid = jnp.logical_and(yi > 0, xi > 0)
        fi = jnp.maximum(xi - 1, 0) * _W + jnp.maximum(yi - 1, 0)
        wk = jnp.where(valid, sign * invc, 0.0)
        idx_ref[:, k * _NR:(k + 1) * _NR] = (
            jnp.broadcast_to(fi, (_RHW, _RHW, _NR)).reshape(_NB, _NR))
        wgt_ref[:, k * _NR:(k + 1) * _NR] = wk.reshape(_NB, _NR)


def _sc_pool_body(ii_hbm, idx_hbm, wgt_hbm, out_hbm, ii_v, idx_v, wgt_v, out_v):
    wid = lax.axis_index("s") * _NCORES + lax.axis_index("c")

    def body(k, carry):
        c = k * _NWORK + wid

        @pl.when(c < _C)
        def _():
            pltpu.sync_copy(ii_hbm.at[c], ii_v)
            b = lax.rem(c, _NB)
            pltpu.sync_copy(idx_hbm.at[b], idx_v)
            pltpu.sync_copy(wgt_hbm.at[b], wgt_v)
            for r0 in range(0, _NR, _LANES):
                acc = jnp.zeros((_LANES,), jnp.float32)
                for k4 in range(4):
                    iv = idx_v[pl.ds(k4 * _NR + r0, _LANES)]
                    g = plsc.load_gather(ii_v, [iv])
                    acc = acc + g * wgt_v[pl.ds(k4 * _NR + r0, _LANES)]
                out_v[pl.ds(r0, _LANES)] = acc
            pltpu.sync_copy(out_v, out_hbm.at[c])

        return carry

    lax.fori_loop(0, _CPW, body, 0)


@functools.cache
def _sc_pool():
    mesh = plsc.VectorSubcoreMesh(
        core_axis_name="c", subcore_axis_name="s",
        num_cores=_NCORES, num_subcores=_NSUB)
    return pl.kernel(
        _sc_pool_body,
        out_type=jax.ShapeDtypeStruct((_C, _NR), jnp.float32),
        mesh=mesh,
        compiler_params=pltpu.CompilerParams(needs_layout_passes=False),
        scratch_types=[
            pltpu.VMEM((_W * _H,), jnp.float32),
            pltpu.VMEM((4 * _NR,), jnp.int32),
            pltpu.VMEM((4 * _NR,), jnp.float32),
            pltpu.VMEM((_NR,), jnp.float32),
        ],
    )


def _integral_images(FM):
    return pl.pallas_call(
        _ii_body,
        grid=(49,),
        in_specs=[pl.BlockSpec((_NT, _H, _W), lambda i: (i, 0, 0))],
        out_specs=pl.BlockSpec((1, _NT * _W, _H), lambda i: (i, 0, 0)),
        out_shape=jax.ShapeDtypeStruct((49, _NT * _W, _H), jnp.float32),
    )(FM).reshape(_C, _W * _H)


def _corner_tables(rois):
    return pl.pallas_call(
        _idx_body,
        out_shape=(
            jax.ShapeDtypeStruct((_NB, 4 * _NR), jnp.int32),
            jax.ShapeDtypeStruct((_NB, 4 * _NR), jnp.float32),
        ),
    )(jnp.transpose(rois))


def kernel(FM, rois):
    ii = _integral_images(FM)
    idxs, wgts = _corner_tables(rois)
    out_cr = _sc_pool()(ii, idxs, wgts)
    return out_cr.reshape(_NT, _RHW, _RHW, _NR).transpose(3, 0, 1, 2)


# speedup vs baseline: 4.8676x; 1.0426x over previous
import functools

import jax
import jax.numpy as jnp
from jax import lax
from jax.experimental import pallas as pl
from jax.experimental.pallas import tpu as pltpu
from jax.experimental.pallas import tpu_sc as plsc

_NT = 21
_RHW = 7
_H = 64
_W = 64
_C = _NT * _RHW * _RHW
_NB = _RHW * _RHW
_NR = 128
_LANES = 16
_NCORES = 2
_NSUB = 16
_NWORK = _NCORES * _NSUB
_CPW = -(-_C // _NWORK)


def _ii_body(fm_ref, out_ref):
    cb = fm_ref.shape[0]
    a = fm_ref[...]
    r = lax.broadcasted_iota(jnp.int32, (_W, _W), 0)
    s = lax.broadcasted_iota(jnp.int32, (_W, _W), 1)
    u = (r <= s).astype(jnp.float32)
    b1 = jnp.dot(a.reshape(cb * _H, _W), u,
                 precision=lax.Precision.HIGHEST).reshape(cb, _H, _W)
    b1t = jnp.swapaxes(b1, 1, 2)
    b2 = jnp.dot(b1t.reshape(cb * _W, _H), u,
                 precision=lax.Precision.HIGHEST)
    out_ref[...] = b2.reshape(1, cb * _W, _H)


def _idx_body(roist_ref, idx_ref, wgt_ref):
    x1 = jnp.round(roist_ref[0:1, :])
    y1 = jnp.round(roist_ref[1:2, :])
    x2 = jnp.round(roist_ref[2:3, :])
    y2 = jnp.round(roist_ref[3:4, :])
    bin_w = jnp.maximum(x2 - x1 + 1.0, 1.0) / _RHW
    bin_h = jnp.maximum(y2 - y1 + 1.0, 1.0) / _RHW
    p = lax.broadcasted_iota(jnp.int32, (_RHW, _NR), 0).astype(jnp.float32)
    hs = jnp.clip(jnp.floor(p * bin_h) + y1, 0.0, float(_H))
    he = jnp.clip(jnp.ceil((p + 1.0) * bin_h) + y1, 0.0, float(_H))
    ws = jnp.clip(jnp.floor(p * bin_w) + x1, 0.0, float(_W))
    we = jnp.clip(jnp.ceil((p + 1.0) * bin_w) + x1, 0.0, float(_W))
    invc = 1.0 / jnp.maximum((he - hs)[:, None, :] * (we - ws)[None, :, :], 1.0)
    corners = ((he, we, 1.0), (hs, we, -1.0), (he, ws, -1.0), (hs, ws, 1.0))
    for k, (ya, xb, sign) in enumerate(corners):
        yi = ya.astype(jnp.int32)[:, None, :]
        xi = xb.astype(jnp.int32)[None, :, :]
        valid = jnp.logical_and(yi > 0, xi > 0)
        fi = jnp.maximum(xi - 1, 0) * _W + jnp.maximum(yi - 1, 0)
        wk = jnp.where(valid, sign * invc, 0.0)
        idx_ref[:, k * _NR:(k + 1) * _NR] = (
            jnp.broadcast_to(fi, (_RHW, _RHW, _NR)).reshape(_NB, _NR))
        wgt_ref[:, k * _NR:(k + 1) * _NR] = wk.reshape(_NB, _NR)


def _sc_pool_body(ii_hbm, idx_hbm, wgt_hbm, out_hbm, ii_v, idx_v, wgt_v, out_v):
    wid = lax.axis_index("s") * _NCORES + lax.axis_index("c")

    def body(k, carry):
        c = k * _NWORK + wid

        @pl.when(c < _C)
        def _():
            pltpu.sync_copy(ii_hbm.at[c], ii_v)
            b = lax.rem(c, _NB)
            pltpu.sync_copy(idx_hbm.at[b], idx_v)
            pltpu.sync_copy(wgt_hbm.at[b], wgt_v)
            for r0 in range(0, _NR, _LANES):
                acc = jnp.zeros((_LANES,), jnp.float32)
                for k4 in range(4):
                    iv = idx_v[pl.ds(k4 * _NR + r0, _LANES)]
                    g = plsc.load_gather(ii_v, [iv])
                    acc = acc + g * wgt_v[pl.ds(k4 * _NR + r0, _LANES)]
                out_v[pl.ds(r0, _LANES)] = acc
            pltpu.sync_copy(out_v, out_hbm.at[c])

        return carry

    lax.fori_loop(0, _CPW, body, 0)


@functools.cache
def _sc_pool():
    mesh = plsc.VectorSubcoreMesh(
        core_axis_name="c", subcore_axis_name="s",
        num_cores=_NCORES, num_subcores=_NSUB)
    return pl.kernel(
        _sc_pool_body,
        out_type=jax.ShapeDtypeStruct((_C, _NR), jnp.float32),
        mesh=mesh,
        compiler_params=pltpu.CompilerParams(needs_layout_passes=False),
        scratch_types=[
            pltpu.VMEM((_W * _H,), jnp.float32),
            pltpu.VMEM((4 * _NR,), jnp.int32),
            pltpu.VMEM((4 * _NR,), jnp.float32),
            pltpu.VMEM((_NR,), jnp.float32),
        ],
    )


def _integral_images(FM):
    return pl.pallas_call(
        _ii_body,
        grid=(7,),
        in_specs=[pl.BlockSpec((_NT * _RHW, _H, _W), lambda i: (i, 0, 0))],
        out_specs=pl.BlockSpec((1, _NT * _RHW * _W, _H), lambda i: (i, 0, 0)),
        out_shape=jax.ShapeDtypeStruct((7, _NT * _RHW * _W, _H), jnp.float32),
    )(FM).reshape(_C, _W * _H)


def _corner_tables(rois):
    return pl.pallas_call(
        _idx_body,
        out_shape=(
            jax.ShapeDtypeStruct((_NB, 4 * _NR), jnp.int32),
            jax.ShapeDtypeStruct((_NB, 4 * _NR), jnp.float32),
        ),
    )(jnp.transpose(rois))


def kernel(FM, rois):
    ii = _integral_images(FM)
    idxs, wgts = _corner_tables(rois)
    out_cr = _sc_pool()(ii, idxs, wgts)
    return out_cr.reshape(_NT, _RHW, _RHW, _NR).transpose(3, 0, 1, 2)


# speedup vs baseline: 5.3192x; 1.0928x over previous
import functools

import jax
import jax.numpy as jnp
from jax import lax
from jax.experimental import pallas as pl
from jax.experimental.pallas import tpu as pltpu
from jax.experimental.pallas import tpu_sc as plsc

_NT = 21
_RHW = 7
_H = 64
_W = 64
_C = _NT * _RHW * _RHW
_NB = _RHW * _RHW
_NR = 128
_LANES = 16
_NCORES = 2
_NSUB = 16
_NWORK = _NCORES * _NSUB
_CPW = -(-_C // _NWORK)


def _ii_body(fm_ref, out_ref):
    cb = fm_ref.shape[0]
    a = fm_ref[...]
    r = lax.broadcasted_iota(jnp.int32, (_W, _W), 0)
    s = lax.broadcasted_iota(jnp.int32, (_W, _W), 1)
    u = (r <= s).astype(jnp.float32)
    b1 = jnp.dot(a.reshape(cb * _H, _W), u,
                 precision=lax.Precision.HIGHEST).reshape(cb, _H, _W)
    b1t = jnp.swapaxes(b1, 1, 2)
    b2 = jnp.dot(b1t.reshape(cb * _W, _H), u,
                 precision=lax.Precision.HIGHEST)
    out_ref[...] = b2.reshape(cb, _W, _H)


def _idx_body(roist_ref, idx_ref, wgt_ref):
    x1 = jnp.round(roist_ref[0:1, :])
    y1 = jnp.round(roist_ref[1:2, :])
    x2 = jnp.round(roist_ref[2:3, :])
    y2 = jnp.round(roist_ref[3:4, :])
    bin_w = jnp.maximum(x2 - x1 + 1.0, 1.0) / _RHW
    bin_h = jnp.maximum(y2 - y1 + 1.0, 1.0) / _RHW
    p = lax.broadcasted_iota(jnp.int32, (_RHW, _NR), 0).astype(jnp.float32)
    hs = jnp.clip(jnp.floor(p * bin_h) + y1, 0.0, float(_H))
    he = jnp.clip(jnp.ceil((p + 1.0) * bin_h) + y1, 0.0, float(_H))
    ws = jnp.clip(jnp.floor(p * bin_w) + x1, 0.0, float(_W))
    we = jnp.clip(jnp.ceil((p + 1.0) * bin_w) + x1, 0.0, float(_W))
    invc = 1.0 / jnp.maximum((he - hs)[:, None, :] * (we - ws)[None, :, :], 1.0)
    corners = ((he, we, 1.0), (hs, we, -1.0), (he, ws, -1.0), (hs, ws, 1.0))
    for k, (ya, xb, sign) in enumerate(corners):
        yi = ya.astype(jnp.int32)[:, None, :]
        xi = xb.astype(jnp.int32)[None, :, :]
        valid = jnp.logical_and(yi > 0, xi > 0)
        fi = jnp.maximum(xi - 1, 0) * _W + jnp.maximum(yi - 1, 0)
        wk = jnp.where(valid, sign * invc, 0.0)
        idx_ref[:, k * _NR:(k + 1) * _NR] = (
            jnp.broadcast_to(fi, (_RHW, _RHW, _NR)).reshape(_NB, _NR))
        wgt_ref[:, k * _NR:(k + 1) * _NR] = wk.reshape(_NB, _NR)


def _sc_pool_body(ii_hbm, idx_hbm, wgt_hbm, out_hbm, ii_v, idx_v, wgt_v, out_v):
    wid = lax.axis_index("s") * _NCORES + lax.axis_index("c")

    def body(k, carry):
        c = k * _NWORK + wid

        @pl.when(c < _C)
        def _():
            pltpu.sync_copy(ii_hbm.at[c], ii_v)
            b = lax.rem(c, _NB)
            pltpu.sync_copy(idx_hbm.at[b], idx_v)
            pltpu.sync_copy(wgt_hbm.at[b], wgt_v)
            for r0 in range(0, _NR, _LANES):
                acc = jnp.zeros((_LANES,), jnp.float32)
                for k4 in range(4):
                    iv = idx_v[pl.ds(k4 * _NR + r0, _LANES)]
                    ix = lax.shift_right_logical(iv, 6)
                    iy = lax.bitwise_and(iv, 63)
                    g = plsc.load_gather(ii_v, [ix, iy])
                    acc = acc + g * wgt_v[pl.ds(k4 * _NR + r0, _LANES)]
                out_v[pl.ds(r0, _LANES)] = acc
            pltpu.sync_copy(out_v, out_hbm.at[c])

        return carry

    lax.fori_loop(0, _CPW, body, 0)


@functools.cache
def _sc_pool():
    mesh = plsc.VectorSubcoreMesh(
        core_axis_name="c", subcore_axis_name="s",
        num_cores=_NCORES, num_subcores=_NSUB)
    return pl.kernel(
        _sc_pool_body,
        out_type=jax.ShapeDtypeStruct((_C, _NR), jnp.float32),
        mesh=mesh,
        compiler_params=pltpu.CompilerParams(needs_layout_passes=False),
        scratch_types=[
            pltpu.VMEM((_W, _H), jnp.float32),
            pltpu.VMEM((4 * _NR,), jnp.int32),
            pltpu.VMEM((4 * _NR,), jnp.float32),
            pltpu.VMEM((_NR,), jnp.float32),
        ],
    )


def _integral_images(FM):
    return pl.pallas_call(
        _ii_body,
        grid=(7,),
        in_specs=[pl.BlockSpec((_NT * _RHW, _H, _W), lambda i: (i, 0, 0))],
        out_specs=pl.BlockSpec((_NT * _RHW, _W, _H), lambda i: (i, 0, 0)),
        out_shape=jax.ShapeDtypeStruct((_C, _W, _H), jnp.float32),
    )(FM)


def _corner_tables(rois):
    return pl.pallas_call(
        _idx_body,
        out_shape=(
            jax.ShapeDtypeStruct((_NB, 4 * _NR), jnp.int32),
            jax.ShapeDtypeStruct((_NB, 4 * _NR), jnp.float32),
        ),
    )(jnp.transpose(rois))


def kernel(FM, rois):
    ii = _integral_images(FM)
    idxs, wgts = _corner_tables(rois)
    out_cr = _sc_pool()(ii, idxs, wgts)
    return out_cr.reshape(_NT, _RHW, _RHW, _NR).transpose(3, 0, 1, 2)


# speedup vs baseline: 7.6286x; 1.4342x over previous
import functools

import jax
import jax.numpy as jnp
from jax import lax
from jax.experimental import pallas as pl
from jax.experimental.pallas import tpu as pltpu
from jax.experimental.pallas import tpu_sc as plsc

_NT = 21
_RHW = 7
_H = 64
_W = 64
_C = _NT * _RHW * _RHW
_NB = _RHW * _RHW
_NR = 128
_LANES = 16
_NCORES = 2
_NSUB = 16
_NWORK = _NCORES * _NSUB
_CPW = -(-_C // _NWORK)


def _ii_body(fm_ref, out_ref):
    cb = fm_ref.shape[0]
    a = fm_ref[...]
    r = lax.broadcasted_iota(jnp.int32, (_W, _W), 0)
    s = lax.broadcasted_iota(jnp.int32, (_W, _W), 1)
    u = (r <= s).astype(jnp.bfloat16)

    def tri_cumsum(m):
        hi = m.astype(jnp.bfloat16)
        lo = (m - hi.astype(jnp.float32)).astype(jnp.bfloat16)
        return (jnp.dot(hi, u, preferred_element_type=jnp.float32)
                + jnp.dot(lo, u, preferred_element_type=jnp.float32))

    b1 = tri_cumsum(a.reshape(cb * _H, _W)).reshape(cb, _H, _W)
    b1t = jnp.swapaxes(b1, 1, 2)
    b2 = tri_cumsum(b1t.reshape(cb * _W, _H))
    out_ref[...] = b2.reshape(cb, _W, _H)


def _idx_body(roist_ref, idx_ref, wgt_ref):
    x1 = jnp.round(roist_ref[0:1, :])
    y1 = jnp.round(roist_ref[1:2, :])
    x2 = jnp.round(roist_ref[2:3, :])
    y2 = jnp.round(roist_ref[3:4, :])
    bin_w = jnp.maximum(x2 - x1 + 1.0, 1.0) / _RHW
    bin_h = jnp.maximum(y2 - y1 + 1.0, 1.0) / _RHW
    p = lax.broadcasted_iota(jnp.int32, (_RHW, _NR), 0).astype(jnp.float32)
    hs = jnp.clip(jnp.floor(p * bin_h) + y1, 0.0, float(_H))
    he = jnp.clip(jnp.ceil((p + 1.0) * bin_h) + y1, 0.0, float(_H))
    ws = jnp.clip(jnp.floor(p * bin_w) + x1, 0.0, float(_W))
    we = jnp.clip(jnp.ceil((p + 1.0) * bin_w) + x1, 0.0, float(_W))
    invc = 1.0 / jnp.maximum((he - hs)[:, None, :] * (we - ws)[None, :, :], 1.0)
    corners = ((he, we, 1.0), (hs, we, -1.0), (he, ws, -1.0), (hs, ws, 1.0))
    for k, (ya, xb, sign) in enumerate(corners):
        yi = ya.astype(jnp.int32)[:, None, :]
        xi = xb.astype(jnp.int32)[None, :, :]
        valid = jnp.logical_and(yi > 0, xi > 0)
        fi = jnp.maximum(xi - 1, 0) * _W + jnp.maximum(yi - 1, 0)
        wk = jnp.where(valid, sign * invc, 0.0)
        idx_ref[:, k * _NR:(k + 1) * _NR] = (
            jnp.broadcast_to(fi, (_RHW, _RHW, _NR)).reshape(_NB, _NR))
        wgt_ref[:, k * _NR:(k + 1) * _NR] = wk.reshape(_NB, _NR)


def _sc_pool_body(ii_hbm, idx_hbm, wgt_hbm, out_hbm, ii_v, idx_v, wgt_v, out_v):
    wid = lax.axis_index("s") * _NCORES + lax.axis_index("c")

    def body(k, carry):
        c = k * _NWORK + wid

        @pl.when(c < _C)
        def _():
            pltpu.sync_copy(ii_hbm.at[c], ii_v)
            b = lax.rem(c, _NB)
            pltpu.sync_copy(idx_hbm.at[b], idx_v)
            pltpu.sync_copy(wgt_hbm.at[b], wgt_v)
            for r0 in range(0, _NR, _LANES):
                acc = jnp.zeros((_LANES,), jnp.float32)
                for k4 in range(4):
                    iv = idx_v[pl.ds(k4 * _NR + r0, _LANES)]
                    ix = lax.shift_right_logical(iv, 6)
                    iy = lax.bitwise_and(iv, 63)
                    g = plsc.load_gather(ii_v, [ix, iy])
                    acc = acc + g * wgt_v[pl.ds(k4 * _NR + r0, _LANES)]
                out_v[pl.ds(r0, _LANES)] = acc
            pltpu.sync_copy(out_v, out_hbm.at[c])

        return carry

    lax.fori_loop(0, _CPW, body, 0)


@functools.cache
def _sc_pool():
    mesh = plsc.VectorSubcoreMesh(
        core_axis_name="c", subcore_axis_name="s",
        num_cores=_NCORES, num_subcores=_NSUB)
    return pl.kernel(
        _sc_pool_body,
        out_type=jax.ShapeDtypeStruct((_C, _NR), jnp.float32),
        mesh=mesh,
        compiler_params=pltpu.CompilerParams(needs_layout_passes=False),
        scratch_types=[
            pltpu.VMEM((_W, _H), jnp.float32),
            pltpu.VMEM((4 * _NR,), jnp.int32),
            pltpu.VMEM((4 * _NR,), jnp.float32),
            pltpu.VMEM((_NR,), jnp.float32),
        ],
    )


def _integral_images(FM):
    return pl.pallas_call(
        _ii_body,
        grid=(7,),
        in_specs=[pl.BlockSpec((_NT * _RHW, _H, _W), lambda i: (i, 0, 0))],
        out_specs=pl.BlockSpec((_NT * _RHW, _W, _H), lambda i: (i, 0, 0)),
        out_shape=jax.ShapeDtypeStruct((_C, _W, _H), jnp.float32),
    )(FM)


def _corner_tables(rois):
    return pl.pallas_call(
        _idx_body,
        out_shape=(
            jax.ShapeDtypeStruct((_NB, 4 * _NR), jnp.int32),
            jax.ShapeDtypeStruct((_NB, 4 * _NR), jnp.float32),
        ),
    )(jnp.transpose(rois))


def kernel(FM, rois):
    ii = _integral_images(FM)
    idxs, wgts = _corner_tables(rois)
    out_cr = _sc_pool()(ii, idxs, wgts)
    return out_cr.reshape(_NT, _RHW, _RHW, _NR).transpose(3, 0, 1, 2)
